# baseline (speedup 1.0000x reference)
import functools

import jax
import jax.numpy as jnp
from jax import lax
from jax.experimental import pallas as pl
from jax.experimental.pallas import tpu as pltpu
from jax.experimental.pallas import tpu_sc as plsc

NUM_CORES = 2
NUM_SUBCORES = 16
NUM_WORKERS = NUM_CORES * NUM_SUBCORES
LANES = 16

BATCH = 16384
EMB_DIM = 64
ROWS_PER_WORKER = BATCH // NUM_WORKERS
CHUNK = 128
NUM_CHUNKS = ROWS_PER_WORKER // CHUNK
BLOCKS_PER_CHUNK = CHUNK // LANES


def _gmf_body(uids_hbm, iids_hbm, utab_hbm, itab_hbm, wb_hbm, b_hbm,
              out_hbm, uidx_v, iidx_v, urows_v, irows_v, b_v, wb_v,
              out_v, usem, isem):
    wid = lax.axis_index("s") * NUM_CORES + lax.axis_index("c")
    base = wid * ROWS_PER_WORKER

    pltpu.sync_copy(uids_hbm.at[pl.ds(base, ROWS_PER_WORKER)], uidx_v)
    pltpu.sync_copy(iids_hbm.at[pl.ds(base, ROWS_PER_WORKER)], iidx_v)
    pltpu.sync_copy(wb_hbm, wb_v)
    pltpu.sync_copy(b_hbm, b_v)

    bvec = b_v[:]
    iota = lax.iota(jnp.int32, LANES)

    def fire(blk, carry):
        uvec = uidx_v[pl.ds(blk * LANES, LANES)]
        ivec = iidx_v[pl.ds(blk * LANES, LANES)]
        dst = (blk % BLOCKS_PER_CHUNK) * LANES
        for j in range(LANES):
            pltpu.async_copy(
                utab_hbm.at[uvec[j]], urows_v.at[dst + j], usem)
            pltpu.async_copy(
                itab_hbm.at[ivec[j]], irows_v.at[dst + j], isem)
        return carry

    def compute(blk, carry):
        rows = (blk % BLOCKS_PER_CHUNK) * LANES + iota
        acc = jnp.zeros((LANES,), jnp.float32)
        for d in range(EMB_DIM):
            col = jnp.full((LANES,), d, jnp.int32)
            ucol = plsc.load_gather(urows_v, [rows, col])
            vcol = plsc.load_gather(irows_v, [rows, col])
            acc = acc + ucol * vcol * wb_v[pl.ds(d * LANES, LANES)]
        logits = acc + bvec
        out_v[pl.ds(blk * LANES, LANES)] = 1.0 / (1.0 + jnp.exp(-logits))
        return carry

    for ch in range(NUM_CHUNKS):
        lax.fori_loop(ch * BLOCKS_PER_CHUNK, (ch + 1) * BLOCKS_PER_CHUNK,
                      fire, 0)
        pltpu.make_async_copy(
            utab_hbm.at[pl.ds(0, CHUNK)], urows_v, usem).wait()
        pltpu.make_async_copy(
            itab_hbm.at[pl.ds(0, CHUNK)], irows_v, isem).wait()
        lax.fori_loop(ch * BLOCKS_PER_CHUNK, (ch + 1) * BLOCKS_PER_CHUNK,
                      compute, 0)

    pltpu.sync_copy(out_v, out_hbm.at[pl.ds(base, ROWS_PER_WORKER)])


_gmf_kernel = functools.partial(
    pl.kernel,
    out_type=jax.ShapeDtypeStruct((BATCH,), jnp.float32),
    mesh=plsc.VectorSubcoreMesh(
        core_axis_name="c", subcore_axis_name="s",
        num_cores=NUM_CORES, num_subcores=NUM_SUBCORES),
    compiler_params=pltpu.CompilerParams(needs_layout_passes=False),
    scratch_types=[
        pltpu.VMEM((ROWS_PER_WORKER,), jnp.int32),
        pltpu.VMEM((ROWS_PER_WORKER,), jnp.int32),
        pltpu.VMEM((CHUNK, EMB_DIM), jnp.float32),
        pltpu.VMEM((CHUNK, EMB_DIM), jnp.float32),
        pltpu.VMEM((LANES,), jnp.float32),
        pltpu.VMEM((EMB_DIM * LANES,), jnp.float32),
        pltpu.VMEM((ROWS_PER_WORKER,), jnp.float32),
        pltpu.SemaphoreType.DMA,
        pltpu.SemaphoreType.DMA,
    ],
)(_gmf_body)


@jax.jit
def kernel(user_ids, item_ids, user_table, item_table, W, b):
    uids = user_ids.astype(jnp.int32)
    iids = item_ids.astype(jnp.int32)
    wb = jnp.broadcast_to(W.reshape(EMB_DIM, 1).astype(jnp.float32),
                          (EMB_DIM, LANES)).reshape(EMB_DIM * LANES)
    b16 = jnp.broadcast_to(b.astype(jnp.float32), (LANES,))
    out = _gmf_kernel(uids, iids, user_table, item_table, wb, b16)
    return out.reshape(BATCH, 1)
